# baseline (speedup 1.0000x reference)
import jax
import jax.numpy as jnp
from jax import lax
from jax.experimental import pallas as pl

_B, _T, _D = 8, 1024, 1024
_CD, _V = 16, 8192
_TB = 256


def _vq_body(x_ref, p_ref, cbt_ref, out_ref):
    xp = lax.dot_general(
        x_ref[...], p_ref[...], (((1,), (0,)), ((), ())),
        preferred_element_type=jnp.float32,
    )
    norm = jnp.sqrt(jnp.sum(xp * xp, axis=1, keepdims=True))
    xn = xp / jnp.clip(norm, 1e-12, None)
    cbt = cbt_ref[...]
    csq = jnp.sum(cbt * cbt, axis=0, keepdims=True)
    dots2 = lax.dot_general(
        xn * -2.0, cbt, (((1,), (0,)), ((), ())),
        preferred_element_type=jnp.float32,
    )
    s = dots2 + csq
    acc_v = s[:, 0:128]
    acc_j = jnp.zeros((_TB, 128), jnp.int32)
    for j in range(1, _V // 128):
        v = s[:, j * 128:(j + 1) * 128]
        pred = v < acc_v
        acc_v = jnp.where(pred, v, acc_v)
        acc_j = jnp.where(pred, j, acc_j)
    lane = lax.broadcasted_iota(jnp.int32, (_TB, 128), 1)
    linear = acc_j * 128 + lane
    m = jnp.min(acc_v, axis=1, keepdims=True)
    cand = jnp.where(acc_v == m, linear, jnp.int32(_V))
    out_ref[...] = jnp.min(cand, axis=1)[:, None]


def kernel(x, P, CB):
    x2 = x.reshape(_B * _T, _D)
    cbt = CB.T
    out = pl.pallas_call(
        _vq_body,
        grid=(_B * _T // _TB,),
        in_specs=[
            pl.BlockSpec((_TB, _D), lambda i: (i, 0)),
            pl.BlockSpec((_D, _CD), lambda i: (0, 0)),
            pl.BlockSpec((_CD, _V), lambda i: (0, 0)),
        ],
        out_specs=pl.BlockSpec((_TB, 1), lambda i: (i, 0)),
        out_shape=jax.ShapeDtypeStruct((_B * _T, 1), jnp.int32),
    )(x2, P, cbt)
    return out.reshape(_B, _T)


# speedup vs baseline: 1.1809x; 1.1809x over previous
import jax
import jax.numpy as jnp
from jax import lax
from jax.experimental import pallas as pl

_B, _T, _D = 8, 1024, 1024
_CD, _V = 16, 8192
_TB = 1024


def _vq_body(x_ref, p_ref, cbt_ref, out_ref):
    xp = lax.dot_general(
        x_ref[...], p_ref[...], (((1,), (0,)), ((), ())),
        preferred_element_type=jnp.float32,
    )
    norm = jnp.sqrt(jnp.sum(xp * xp, axis=1, keepdims=True))
    xn = xp / jnp.clip(norm, 1e-12, None)
    cbt = cbt_ref[...]
    csq = jnp.sum(cbt * cbt, axis=0, keepdims=True)
    dots2 = lax.dot_general(
        xn * -2.0, cbt, (((1,), (0,)), ((), ())),
        preferred_element_type=jnp.float32,
    )
    s = dots2 + csq
    acc_v = s[:, 0:128]
    acc_j = jnp.zeros((_TB, 128), jnp.int32)
    for j in range(1, _V // 128):
        v = s[:, j * 128:(j + 1) * 128]
        pred = v < acc_v
        acc_v = jnp.where(pred, v, acc_v)
        acc_j = jnp.where(pred, j, acc_j)
    lane = lax.broadcasted_iota(jnp.int32, (_TB, 128), 1)
    linear = acc_j * 128 + lane
    m = jnp.min(acc_v, axis=1, keepdims=True)
    cand = jnp.where(acc_v == m, linear, jnp.int32(_V))
    out_ref[...] = jnp.min(cand, axis=1)[:, None]


def kernel(x, P, CB):
    x2 = x.reshape(_B * _T, _D)
    cbt = CB.T
    out = pl.pallas_call(
        _vq_body,
        grid=(_B * _T // _TB,),
        in_specs=[
            pl.BlockSpec((_TB, _D), lambda i: (i, 0)),
            pl.BlockSpec((_D, _CD), lambda i: (0, 0)),
            pl.BlockSpec((_CD, _V), lambda i: (0, 0)),
        ],
        out_specs=pl.BlockSpec((_TB, 1), lambda i: (i, 0)),
        out_shape=jax.ShapeDtypeStruct((_B * _T, 1), jnp.int32),
    )(x2, P, cbt)
    return out.reshape(_B, _T)


# speedup vs baseline: 1.2029x; 1.0186x over previous
import jax
import jax.numpy as jnp
from jax import lax
from jax.experimental import pallas as pl

_B, _T, _D = 8, 1024, 1024
_CD, _V = 16, 8192
_TB = 1024
_GR = 64


def _vq_body(x_ref, p_ref, cbt_ref, out_ref):
    xp = lax.dot_general(
        x_ref[...], p_ref[...], (((1,), (0,)), ((), ())),
        preferred_element_type=jnp.float32,
    )
    norm = jnp.sqrt(jnp.sum(xp * xp, axis=1, keepdims=True))
    xn = xp / jnp.clip(norm, 1e-12, None)
    cbt = cbt_ref[...]
    csq = jnp.sum(cbt * cbt, axis=0, keepdims=True)
    dots2 = lax.dot_general(
        xn * -2.0, cbt, (((1,), (0,)), ((), ())),
        preferred_element_type=jnp.float32,
    )
    lane = lax.broadcasted_iota(jnp.int32, (_GR, 128), 1)
    outs = []
    for g in range(_TB // _GR):
        sg = dots2[g * _GR:(g + 1) * _GR, :]
        acc_v = sg[:, 0:128] + csq[:, 0:128]
        acc_j = jnp.zeros((_GR, 128), jnp.int32)
        for j in range(1, _V // 128):
            v = sg[:, j * 128:(j + 1) * 128] + csq[:, j * 128:(j + 1) * 128]
            pred = v < acc_v
            acc_v = jnp.minimum(acc_v, v)
            acc_j = jnp.where(pred, j, acc_j)
        linear = acc_j * 128 + lane
        m = jnp.min(acc_v, axis=1, keepdims=True)
        cand = jnp.where(acc_v == m, linear, jnp.int32(_V))
        outs.append(jnp.min(cand, axis=1)[:, None])
    out_ref[...] = jnp.concatenate(outs, axis=0)


def kernel(x, P, CB):
    x2 = x.reshape(_B * _T, _D)
    cbt = CB.T
    out = pl.pallas_call(
        _vq_body,
        grid=(_B * _T // _TB,),
        in_specs=[
            pl.BlockSpec((_TB, _D), lambda i: (i, 0)),
            pl.BlockSpec((_D, _CD), lambda i: (0, 0)),
            pl.BlockSpec((_CD, _V), lambda i: (0, 0)),
        ],
        out_specs=pl.BlockSpec((_TB, 1), lambda i: (i, 0)),
        out_shape=jax.ShapeDtypeStruct((_B * _T, 1), jnp.int32),
    )(x2, P, cbt)
    return out.reshape(_B, _T)
